# Initial kernel scaffold; baseline (speedup 1.0000x reference)
import jax
import jax.numpy as jnp
from jax.experimental import pallas as pl


def kernel(node_features, edge_features, edge_list, W0, Wn0, We0, Wl, Wnl, Wel):
    raise NotImplementedError("write your pallas kernel here")


# speedup vs baseline: 3.3098x; 3.3098x over previous
import functools

import jax
import jax.numpy as jnp
from jax import lax
from jax.experimental import pallas as pl
from jax.experimental.pallas import tpu as pltpu
from jax.experimental.pallas import tpu_sc as plsc

N = 2048
P = 4
C = 16
OUT = P * C
E = 65536
ALPHA = 0.1

NC = 2
NS = 16
NW = NC * NS
EPW = E // NW
WIN = 128
NWIN = EPW // WIN
TROW = 128
RPT = N // NS

f32 = jnp.float32
i32 = jnp.int32

_mesh = plsc.VectorSubcoreMesh(core_axis_name="c", subcore_axis_name="s",
                               num_cores=NC, num_subcores=NS)
_SC_PARAMS = pltpu.CompilerParams(needs_layout_passes=False)


def _iota16():
  return lax.iota(i32, 16)


def _splat_i(v):
  return jnp.full((16,), v, i32)


def _xn_t(xv, wn):
  return lax.dot_general(wn, xv, (((0,), (1,)), ((), ())),
                         preferred_element_type=f32)


def _agg_sum(a):
  blocks = [sum(a[c, 4 * b + g] for c in range(NC) for g in range(4))
            for b in range(4)]
  return jnp.concatenate(blocks, axis=1)


def _agg4d(agg):
  return agg.reshape(NC, NS, N, 16)


def _tc_entry(x, wy, wn):
  def body(x_r, wy_r, wn_r, t_r, xnt_r):
    t_r[...] = jnp.dot(x_r[...], wy_r[...], preferred_element_type=f32)
    xnt_r[...] = _xn_t(x_r[...], wn_r[...])

  return pl.pallas_call(
      body,
      out_shape=[jax.ShapeDtypeStruct((N, TROW), f32),
                 jax.ShapeDtypeStruct((P, N), f32)],
  )(x, wy, wn)


def _tc_stage(agg, r1, r2, wy, wn):
  def body(a_r, r1_r, r2_r, wy_r, wn_r, x_r, t_r, xnt_r):
    xv = jax.nn.relu((1.0 - ALPHA) * _agg_sum(a_r[...])
                     + ALPHA * (r1_r[...] + r2_r[...]))
    x_r[...] = xv
    t_r[...] = jnp.dot(xv, wy_r[...], preferred_element_type=f32)
    xnt_r[...] = _xn_t(xv, wn_r[...])

  return pl.pallas_call(
      body,
      out_shape=[jax.ShapeDtypeStruct((N, OUT), f32),
                 jax.ShapeDtypeStruct((N, TROW), f32),
                 jax.ShapeDtypeStruct((P, N), f32)],
  )(agg, r1, r2, wy, wn)


def _tc_final(agg, r1, r2):
  def body(a_r, r1_r, r2_r, x_r):
    x_r[...] = jax.nn.relu((1.0 - ALPHA) * _agg_sum(a_r[...])
                           + ALPHA * (r1_r[...] + r2_r[...]))

  return pl.pallas_call(
      body, out_shape=jax.ShapeDtypeStruct((N, OUT), f32),
  )(agg, r1, r2)


def _tc_copy(ef128):
  def body(i_r, o_r):
    o_r[...] = i_r[...]

  rows = N * N * P // 128
  return pl.pallas_call(
      body,
      grid=(16,),
      in_specs=[pl.BlockSpec((rows // 16, 128), lambda i: (i, 0))],
      out_specs=pl.BlockSpec((rows // 16, 128), lambda i: (i, 0)),
      out_shape=jax.ShapeDtypeStruct((rows, 128), f32),
  )(ef128)


ECORE = E // NC
EPT = ECORE // NS
EPG = ECORE // 4


def _sc_layer_call(first, *args):
  scratch = [
      pltpu.VMEM((WIN,), i32),
      pltpu.VMEM((WIN,), i32),
      pltpu.VMEM((WIN, TROW), f32),
      pltpu.VMEM((P * N,), f32),
      pltpu.VMEM((4, WIN), f32),
      pltpu.VMEM((4, WIN), f32),
      pltpu.VMEM((WIN,), f32),
      pltpu.VMEM((16,), f32),
      pltpu.VMEM((N * 16,), f32),
  ]
  if first:
    scratch += [pltpu.VMEM((WIN,), i32),
                pltpu.VMEM((WIN,), i32),
                pltpu.VMEM((WIN, 128), f32)]
  outs = [jax.ShapeDtypeStruct((NC, NS, N * 16), f32),
          jax.ShapeDtypeStruct((4, E), f32)]
  if first:
    outs.append(jax.ShapeDtypeStruct((4, E), f32))

  def body(*refs):
    if first:
      (t_h, xnt_h, src_h, dst_h, we_h, ridx_h, off32_h, ef128_h,
       agg_h, eupd_h, ev0_h,
       idx_s, idx_d, yrows, xn_v, ein_s, eupd_s, eup_b, we_s, acc,
       ridx_s, off32_s, rows128) = refs
    else:
      (t_h, xnt_h, src_h, dst_h, we_h, ein_h,
       agg_h, eupd_h,
       idx_s, idx_d, yrows, xn_v, ein_s, eupd_s, eup_b, we_s, acc) = refs
    c = lax.axis_index("c")
    s = lax.axis_index("s")
    pltpu.sync_copy(we_h, we_s)
    for p in range(P):
      pltpu.sync_copy(xnt_h.at[p], xn_v.at[pl.ds(p * N, N)])
    zv = jnp.zeros((16,), f32)

    @pl.loop(0, N * 16, step=16)
    def _z(r):
      acc[pl.ds(r, 16)] = zv

    @pl.loop(0, EPT // WIN)
    def _win(j):
      base = c * ECORE + s * EPT + j * WIN
      lbase = s * EPT + j * WIN
      pltpu.sync_copy(src_h.at[pl.ds(base, WIN)], idx_s)
      pltpu.sync_copy(dst_h.at[pl.ds(base, WIN)], idx_d)
      if first:
        pltpu.sync_copy(ridx_h.at[pl.ds(base, WIN)], ridx_s)
        pltpu.sync_copy(off32_h.at[pl.ds(base, WIN)], off32_s)
        pltpu.sync_copy(ef128_h.at[ridx_s], rows128)
        for v in range(WIN // 16):
          sl = pl.ds(16 * v, 16)
          maj = _iota16() + (16 * v)
          off = off32_s[sl]
          for q in range(4):
            ein_s[q, sl] = plsc.load_gather(rows128, [maj, off + q])
      else:
        for q in range(4):
          pltpu.sync_copy(ein_h.at[q, pl.ds(base, WIN)], ein_s.at[q])
      wev = we_s[pl.ds(0, 16)]
      for v in range(WIN // 16):
        sl = pl.ds(16 * v, 16)
        srcv = idx_s[sl]
        dstv = idx_d[sl]
        e0 = ein_s[0, sl]
        e1 = ein_s[1, sl]
        e2 = ein_s[2, sl]
        e3 = ein_s[3, sl]
        for p in range(4):
          xns = plsc.load_gather(xn_v, [srcv + _splat_i(p * N)])
          xndv = plsc.load_gather(xn_v, [dstv + _splat_i(p * N)])
          z = (e0 * wev[p] + e1 * wev[4 + p] + e2 * wev[8 + p]
               + e3 * wev[12 + p] + xns + xndv)
          t = jnp.exp(-2.0 * jnp.abs(z))
          r = (1.0 - t) / (1.0 + t)
          eupd_s[p, sl] = jnp.where(z < 0.0, -r, r)
      for q in range(4):
        pltpu.sync_copy(eupd_s.at[q], eupd_h.at[q, pl.ds(base, WIN)])
        if first:
          pltpu.sync_copy(ein_s.at[q], ev0_h.at[q, pl.ds(base, WIN)])

    plsc.subcore_barrier()

    b = s // 4
    g = s % 4
    iota = _iota16()
    bb = jnp.full((16,), 16, i32) * b + iota

    @pl.loop(0, EPG // WIN)
    def _win2(j):
      base = c * ECORE + g * EPG + j * WIN
      pltpu.sync_copy(src_h.at[pl.ds(base, WIN)], idx_s)
      pltpu.sync_copy(dst_h.at[pl.ds(base, WIN)], idx_d)
      pltpu.sync_copy(eupd_h.at[b, pl.ds(base, WIN)], eup_b)
      pltpu.sync_copy(t_h.at[idx_s], yrows)

      @pl.loop(0, WIN)
      def _edge(k):
        kv = jnp.full((16,), k, i32)
        dstv = plsc.load_gather(idx_d, [kv])
        eb = plsc.load_gather(eup_b, [kv])
        val = eb * plsc.load_gather(yrows, [kv, bb])
        plsc.addupdate_scatter(acc, [dstv * 16 + iota], val)

    pltpu.sync_copy(acc, agg_h.at[c, s])

  return pl.kernel(body, out_type=outs, mesh=_mesh, scratch_types=scratch,
                   compiler_params=_SC_PARAMS)(*args)


def _sc_scatter(eref, flat4, eupd):
  scratch = [pltpu.VMEM((WIN,), i32),
             pltpu.VMEM((WIN,), i32),
             pltpu.VMEM((WIN,), f32)]

  @functools.partial(pl.kernel, out_type=(), mesh=_mesh,
                     scratch_types=scratch, compiler_params=_SC_PARAMS)
  def body(e1d, f4_h, eu_h, f4_s, idx_s, pay_s):
    c = lax.axis_index("c")
    s = lax.axis_index("s")
    wid = s * NC + c

    @pl.loop(0, NWIN)
    def _(j):
      base = wid * EPW + j * WIN
      pltpu.sync_copy(f4_h.at[pl.ds(base, WIN)], f4_s)
      for p in range(4):
        for v in range(WIN // 16):
          sl = pl.ds(16 * v, 16)
          idx_s[sl] = f4_s[sl] + p
        pltpu.sync_copy(eu_h.at[p, pl.ds(base, WIN)], pay_s)
        pltpu.sync_copy(pay_s, e1d.at[idx_s])

  body(eref, flat4, eupd)


def _wy(W):
  F = W.shape[1]
  wy = jnp.transpose(W, (1, 0, 2)).reshape(F, P * C)
  return jnp.concatenate([wy, jnp.zeros((F, TROW - OUT), f32)], axis=1)


def kernel(node_features, edge_features, edge_list, W0, Wn0, We0, Wl, Wnl,
           Wel):
  src = edge_list[0].astype(i32)
  dst = edge_list[1].astype(i32)
  flat = src * N + dst
  rowidx = flat // 32
  off32 = (flat % 32) * 4
  flat4 = flat * 4
  e1d_in = edge_features.reshape(-1)
  ef128 = e1d_in.reshape(N * N * P // 128, 128)
  zeros = jnp.zeros((N, OUT), f32)
  nf64 = node_features[:, :OUT]

  t, xnt = _tc_entry(node_features, _wy(W0), Wn0)
  agg, _, ev0 = _sc_layer_call(True, t, xnt, src, dst, We0.reshape(16),
                               rowidx, off32, ef128)
  xa, t, xnt = _tc_stage(_agg4d(agg), nf64, zeros, _wy(Wl[0]), Wnl[0])
  agg, eupd = _sc_layer_call(False, t, xnt, src, dst, Wel[0].reshape(16),
                             ev0)
  xb, t, xnt = _tc_stage(_agg4d(agg), nf64, nf64, _wy(Wl[1]), Wnl[1])
  agg, eupd = _sc_layer_call(False, t, xnt, src, dst, Wel[1].reshape(16),
                             eupd)
  xc, t, xnt = _tc_stage(_agg4d(agg), nf64, xa, _wy(Wl[2]), Wnl[2])
  agg, eupd = _sc_layer_call(False, t, xnt, src, dst, Wel[2].reshape(16),
                             eupd)
  x = _tc_final(_agg4d(agg), nf64, xb)

  e_copy = _tc_copy(ef128)
  eref = jax.new_ref(e_copy.reshape(N * N * P))
  _sc_scatter(eref, flat4, eupd)
  e_out = eref[...].reshape(N, N, P)
  return x, e_out
